# Initial kernel scaffold; baseline (speedup 1.0000x reference)
import jax
import jax.numpy as jnp
from jax.experimental import pallas as pl


def kernel(x, edge_index, num_real_nodes, eps, W1, b1, g1, bt1, W2, b2, g2, bt2, Wmu, bmu, Wlv, blv, Ue1, ce1, Ue2, ce2, Un1, cn1, Un2, cn2):
    raise NotImplementedError("write your pallas kernel here")


# speedup vs baseline: 5.2488x; 5.2488x over previous
import functools

import jax
import jax.numpy as jnp
from jax import lax
from jax.experimental import pallas as pl
from jax.experimental.pallas import tpu as pltpu
from jax.experimental.pallas import tpu_sc as plsc

N = 512
HID = 128
OFF = N * (N - 1) // 2
ROWS_PER_STEP = 8
NSTEPS = N // ROWS_PER_STEP
F32 = jnp.float32


def _build_counts(edge_index):
    n_edges = edge_index.shape[1]
    per_worker = n_edges // 32

    @functools.partial(
        pl.kernel,
        out_type=jax.ShapeDtypeStruct((2, N * N), F32),
        mesh=plsc.VectorSubcoreMesh(core_axis_name="c", subcore_axis_name="s"),
        scratch_types=[
            pltpu.VMEM((per_worker,), jnp.int32),
            pltpu.VMEM((per_worker,), jnp.int32),
            pltpu.VMEM((per_worker,), jnp.int32),
            pltpu.VMEM((per_worker,), F32),
            pltpu.VMEM((2048,), F32),
            pltpu.VMEM_SHARED((N * N,), F32),
        ],
    )
    def k(ei_hbm, out_hbm, src_v, dst_v, idx_v, ones_v, zbuf, c_shared):
        cid = lax.axis_index("c")
        sid = lax.axis_index("s")
        wid = sid * 2 + cid
        base = wid * per_worker
        pltpu.sync_copy(ei_hbm.at[0, pl.ds(base, per_worker)], src_v)
        pltpu.sync_copy(ei_hbm.at[1, pl.ds(base, per_worker)], dst_v)

        @pl.loop(0, per_worker, step=16)
        def _(c):
            d = dst_v[pl.ds(c, 16)]
            s_ = src_v[pl.ds(c, 16)]
            idx_v[pl.ds(c, 16)] = d * N + s_
            ones_v[pl.ds(c, 16)] = jnp.full((16,), 1.0, F32)

        @pl.loop(0, 2048, step=16)
        def _(c):
            zbuf[pl.ds(c, 16)] = jnp.zeros((16,), F32)

        stripe = (N * N) // 16
        for kk in range(stripe // 2048):
            pltpu.sync_copy(zbuf, c_shared.at[pl.ds(sid * stripe + kk * 2048, 2048)])
        plsc.subcore_barrier()
        pltpu.sync_copy(ones_v, c_shared.at[idx_v], add=True)
        plsc.subcore_barrier()
        pltpu.sync_copy(c_shared.at[pl.ds(sid * stripe, stripe)],
                        out_hbm.at[cid, pl.ds(sid * stripe, stripe)])

    return k(edge_index)


WROWS = 640
CROWS = 640


def _window_params(i):
    off = 511 * i - (i * (i - 1)) // 2
    s = off - i - 1
    su = pl.multiple_of(jnp.clip((s // 8) * 8, 0, OFF - WROWS), 8)
    s1 = pl.multiple_of(jnp.clip((s // 128) * 128, 0, OFF - CROWS), 128)
    return s, su, s1


def _tc_body(c2, x, eps, w1, b1, g1, bt1, w2, b2, g2, bt2, wmu, bmu, wlv, blv,
             ue1, ce1, un1, cn1, un2, cn2, nrn, ce2_hbm, ue2_hbm, adj_out,
             he_s, nl_s, ubuf, wbuf, cbuf, wsem, csem):
    t = pl.program_id(0)

    def mm(a, b_):
        return lax.dot_general(a, b_, (((1,), (1,)), ((), ())),
                               precision=lax.Precision.HIGHEST,
                               preferred_element_type=F32)

    def copies(step, slot):
        out = []
        for r in range(ROWS_PER_STEP):
            i = step * ROWS_PER_STEP + r
            _, su, s1 = _window_params(i)
            out.append(pltpu.make_async_copy(
                ue2_hbm.at[pl.ds(su, WROWS)],
                wbuf.at[slot, pl.ds(r * WROWS, WROWS)],
                wsem.at[slot, r]))
            out.append(pltpu.make_async_copy(
                ce2_hbm.at[pl.ds(s1, CROWS)],
                cbuf.at[slot, r],
                csem.at[slot, r]))
        return out

    def issue(step, slot):
        for c in copies(step, slot):
            c.start()

    def wait(step, slot):
        for c in copies(step, slot):
            c.wait()

    @pl.when(t == 0)
    def _prologue():
        issue(t, 0)
        issue(t + 1, 1)
        c = c2[0] + c2[1]
        ri = lax.broadcasted_iota(jnp.int32, (N, N), 0)
        ci = lax.broadcasted_iota(jnp.int32, (N, N), 1)
        c = c + jnp.where(ri == ci, 1.0, 0.0)
        deg = jnp.sum(c, axis=1, keepdims=True)
        dinv = lax.rsqrt(jnp.maximum(deg, 1e-12))

        def conv(h, w_, b_):
            hw = mm(h, w_) + b_
            agg = lax.dot_general(c, hw * dinv, (((1,), (0,)), ((), ())),
                                  precision=lax.Precision.HIGHEST,
                                  preferred_element_type=F32)
            return agg * dinv

        def bn(h, g_, bt_):
            mu = jnp.mean(h, axis=0, keepdims=True)
            var = jnp.mean((h - mu) ** 2, axis=0, keepdims=True)
            return (h - mu) * lax.rsqrt(var + 1e-5) * g_ + bt_

        h1 = jax.nn.relu(bn(conv(x[...], w1[...], b1[...]), g1[...], bt1[...]))
        h2 = jax.nn.relu(bn(conv(h1, w2[...], b2[...]), g2[...], bt2[...]))
        gv = jnp.sum(h2, axis=0, keepdims=True) / nrn[0, 0]
        mu_ = mm(gv, wmu[...]) + bmu[...]
        lv = jnp.clip(mm(gv, wlv[...]) + blv[...], -4.0, 4.0)
        z = mu_ + eps[...] * jnp.exp(0.5 * lv)
        he_s[...] = jax.nn.relu(mm(z, ue1[...]) + ce1[...])
        hn = jax.nn.relu(mm(z, un1[...]) + cn1[...])
        nl_s[...] = mm(hn, un2[...]) + cn2[...]

    @pl.when(t > 0)
    def _steady():
        @pl.when(t < NSTEPS - 1)
        def _():
            issue(t + 1, lax.rem(t + 1, 2))

    slot = lax.rem(t, 2)
    wait(t, slot)
    w = wbuf[slot]
    he = he_s[...]
    out = lax.dot_general(he, w, (((1,), (1,)), ((), ())),
                          precision=lax.Precision.HIGHEST,
                          preferred_element_type=F32)
    col = lax.broadcasted_iota(jnp.int32, (1, N), 1)
    for r in range(ROWS_PER_STEP):
        i = t * ROWS_PER_STEP + r
        s, su, s1 = _window_params(i)
        seg = out[:, r * WROWS:(r + 1) * WROWS]
        rowm = pltpu.roll(seg, jnp.remainder(su - s, WROWS), axis=1)[:, :N]
        rowc = pltpu.roll(jnp.reshape(cbuf[slot, r], (1, CROWS)),
                          jnp.remainder(s1 - s, CROWS), axis=1)[:, :N]
        masked = jnp.where(col > i, rowm + rowc, 0.0)
        ubuf[pl.ds(i, 1), :] = masked

    @pl.when(t == NSTEPS - 1)
    def _finalize():
        u = ubuf[...]
        ri = lax.broadcasted_iota(jnp.int32, (N, N), 0)
        ci = lax.broadcasted_iota(jnp.int32, (N, N), 1)
        d = jnp.where(ri == ci, nl_s[...], 0.0)
        adj_out[...] = u + u.T + d


def _tc_call(c2, x, eps, w1, b1, g1, bt1, w2, b2, g2, bt2, wmu, bmu, wlv, blv,
             ue1, ce1, un1, cn1, un2, cn2, nrn, ce2, ue2):
    full = lambda shape: pl.BlockSpec(shape, lambda t: tuple(0 for _ in shape))
    vmem_args = [c2, x, eps, w1, b1, g1, bt1, w2, b2, g2, bt2, wmu, bmu, wlv,
                 blv, ue1, ce1, un1, cn1, un2, cn2, nrn]
    in_specs = [full(a.shape) for a in vmem_args]
    in_specs += [pl.BlockSpec(memory_space=pl.ANY),
                 pl.BlockSpec(memory_space=pl.ANY)]
    return pl.pallas_call(
        _tc_body,
        grid=(NSTEPS,),
        in_specs=in_specs,
        out_specs=pl.BlockSpec((N, N), lambda t: (0, 0)),
        out_shape=jax.ShapeDtypeStruct((N, N), F32),
        scratch_shapes=[
            pltpu.VMEM((1, HID), F32),
            pltpu.VMEM((1, N), F32),
            pltpu.VMEM((N, N), F32),
            pltpu.VMEM((2, ROWS_PER_STEP * WROWS, HID), F32),
            pltpu.VMEM((2, ROWS_PER_STEP, CROWS), F32),
            pltpu.SemaphoreType.DMA((2, ROWS_PER_STEP)),
            pltpu.SemaphoreType.DMA((2, ROWS_PER_STEP)),
        ],
        compiler_params=pltpu.CompilerParams(
            dimension_semantics=("arbitrary",)),
    )(c2, x, eps, w1, b1, g1, bt1, w2, b2, g2, bt2, wmu, bmu, wlv, blv,
      ue1, ce1, un1, cn1, un2, cn2, nrn, ce2, ue2)


def kernel(x, edge_index, num_real_nodes, eps, W1, b1, g1, bt1, W2, b2, g2,
           bt2, Wmu, bmu, Wlv, blv, Ue1, ce1, Ue2, ce2, Un1, cn1, Un2, cn2):
    c2 = _build_counts(edge_index).reshape(2, N, N)
    nrn = jnp.asarray(num_real_nodes, F32).reshape(1, 1)
    return _tc_call(c2, x, eps, W1, b1, g1, bt1, W2, b2, g2, bt2, Wmu, bmu,
                    Wlv, blv, Ue1, ce1, Un1, cn1, Un2, cn2, nrn, ce2, Ue2)


# speedup vs baseline: 6.2863x; 1.1977x over previous
import functools

import jax
import jax.numpy as jnp
from jax import lax
from jax.experimental import pallas as pl
from jax.experimental.pallas import tpu as pltpu
from jax.experimental.pallas import tpu_sc as plsc

N = 512
HID = 128
OFF = N * (N - 1) // 2
ROWS_PER_STEP = 8
NSTEPS = N // ROWS_PER_STEP
F32 = jnp.float32


def _build_counts(edge_index):
    n_edges = edge_index.shape[1]
    per_worker = n_edges // 32

    @functools.partial(
        pl.kernel,
        out_type=jax.ShapeDtypeStruct((2, N * N), F32),
        mesh=plsc.VectorSubcoreMesh(core_axis_name="c", subcore_axis_name="s"),
        scratch_types=[
            pltpu.VMEM((per_worker,), jnp.int32),
            pltpu.VMEM((per_worker,), jnp.int32),
            pltpu.VMEM((per_worker,), jnp.int32),
            pltpu.VMEM((per_worker,), F32),
            pltpu.VMEM((2048,), F32),
            pltpu.VMEM_SHARED((N * N,), F32),
        ],
    )
    def k(ei_hbm, out_hbm, src_v, dst_v, idx_v, ones_v, zbuf, c_shared):
        cid = lax.axis_index("c")
        sid = lax.axis_index("s")
        wid = sid * 2 + cid
        base = wid * per_worker
        pltpu.sync_copy(ei_hbm.at[0, pl.ds(base, per_worker)], src_v)
        pltpu.sync_copy(ei_hbm.at[1, pl.ds(base, per_worker)], dst_v)

        @pl.loop(0, per_worker, step=16)
        def _(c):
            d = dst_v[pl.ds(c, 16)]
            s_ = src_v[pl.ds(c, 16)]
            idx_v[pl.ds(c, 16)] = d * N + s_
            ones_v[pl.ds(c, 16)] = jnp.full((16,), 1.0, F32)

        @pl.loop(0, 2048, step=16)
        def _(c):
            zbuf[pl.ds(c, 16)] = jnp.zeros((16,), F32)

        stripe = (N * N) // 16
        for kk in range(stripe // 2048):
            pltpu.sync_copy(zbuf, c_shared.at[pl.ds(sid * stripe + kk * 2048, 2048)])
        plsc.subcore_barrier()
        pltpu.sync_copy(ones_v, c_shared.at[idx_v], add=True)
        plsc.subcore_barrier()
        pltpu.sync_copy(c_shared.at[pl.ds(sid * stripe, stripe)],
                        out_hbm.at[cid, pl.ds(sid * stripe, stripe)])

    return k(edge_index)


SPAN = 4224
WIN = 640


def _row_start(i):
    off = 511 * i - (i * (i - 1)) // 2
    return off - i - 1


def _base_start(step):
    s0 = _row_start(step * ROWS_PER_STEP)
    return pl.multiple_of(jnp.clip((s0 // 128) * 128, 0, OFF - SPAN), 128)


def _tc_body(c2, x, eps, w1, b1, g1, bt1, w2, b2, g2, bt2, wmu, bmu, wlv, blv,
             ue1, ce1, un1, cn1, un2, cn2, nrn, ce2_hbm, ue2_hbm, adj_out,
             he_s, nl_s, ubuf, wbuf, cbuf, obuf, wsem, csem):
    t = pl.program_id(0)

    def mm(a, b_):
        return lax.dot_general(a, b_, (((1,), (1,)), ((), ())),
                               precision=lax.Precision.HIGHEST,
                               preferred_element_type=F32)

    def copies(step, slot):
        sb = _base_start(step)
        return [
            pltpu.make_async_copy(ue2_hbm.at[pl.ds(sb, SPAN)],
                                  wbuf.at[slot], wsem.at[slot]),
            pltpu.make_async_copy(ce2_hbm.at[pl.ds(sb, SPAN)],
                                  cbuf.at[slot], csem.at[slot]),
        ]

    def issue(step, slot):
        for c in copies(step, slot):
            c.start()

    def wait(step, slot):
        for c in copies(step, slot):
            c.wait()

    @pl.when(t == 0)
    def _prologue():
        issue(t, 0)
        issue(t + 1, 1)
        c = c2[0] + c2[1]
        ri = lax.broadcasted_iota(jnp.int32, (N, N), 0)
        ci = lax.broadcasted_iota(jnp.int32, (N, N), 1)
        c = c + jnp.where(ri == ci, 1.0, 0.0)
        deg = jnp.sum(c, axis=1, keepdims=True)
        dinv = lax.rsqrt(jnp.maximum(deg, 1e-12))

        def conv(h, w_, b_):
            hw = mm(h, w_) + b_
            agg = lax.dot_general(c, hw * dinv, (((1,), (0,)), ((), ())),
                                  precision=lax.Precision.HIGHEST,
                                  preferred_element_type=F32)
            return agg * dinv

        def bn(h, g_, bt_):
            mu = jnp.mean(h, axis=0, keepdims=True)
            var = jnp.mean((h - mu) ** 2, axis=0, keepdims=True)
            return (h - mu) * lax.rsqrt(var + 1e-5) * g_ + bt_

        h1 = jax.nn.relu(bn(conv(x[...], w1[...], b1[...]), g1[...], bt1[...]))
        h2 = jax.nn.relu(bn(conv(h1, w2[...], b2[...]), g2[...], bt2[...]))
        gv = jnp.sum(h2, axis=0, keepdims=True) / nrn[0, 0]
        mu_ = mm(gv, wmu[...]) + bmu[...]
        lv = jnp.clip(mm(gv, wlv[...]) + blv[...], -4.0, 4.0)
        z = mu_ + eps[...] * jnp.exp(0.5 * lv)
        he_s[...] = jax.nn.relu(mm(z, ue1[...]) + ce1[...])
        hn = jax.nn.relu(mm(z, un1[...]) + cn1[...])
        nl_s[...] = mm(hn, un2[...]) + cn2[...]

    @pl.when(t > 0)
    def _steady():
        @pl.when(t < NSTEPS - 1)
        def _():
            issue(t + 1, lax.rem(t + 1, 2))

    slot = lax.rem(t, 2)
    wait(t, slot)
    w = wbuf[slot]
    he = he_s[...]
    out = lax.dot_general(he, w, (((1,), (1,)), ((), ())),
                          precision=lax.Precision.HIGHEST,
                          preferred_element_type=F32)
    obuf[...] = out
    sb = _base_start(t)
    col = lax.broadcasted_iota(jnp.int32, (1, N), 1)
    for r in range(ROWS_PER_STEP):
        i = t * ROWS_PER_STEP + r
        d = _row_start(i) - sb
        a = pl.multiple_of(jnp.clip((d // 128) * 128, 0, SPAN - WIN), 128)
        b = d - a
        w640 = obuf[:, pl.ds(a, WIN)]
        c640 = jnp.reshape(cbuf[slot, pl.ds(a, WIN)], (1, WIN))
        rolled = pltpu.roll(w640 + c640, jnp.remainder(-b, WIN), axis=1)
        masked = jnp.where(col > i, rolled[:, :N], 0.0)
        ubuf[pl.ds(i, 1), :] = masked

    @pl.when(t == NSTEPS - 1)
    def _finalize():
        u = ubuf[...]
        ri = lax.broadcasted_iota(jnp.int32, (N, N), 0)
        ci = lax.broadcasted_iota(jnp.int32, (N, N), 1)
        d = jnp.where(ri == ci, nl_s[...], 0.0)
        adj_out[...] = u + u.T + d


def _tc_call(c2, x, eps, w1, b1, g1, bt1, w2, b2, g2, bt2, wmu, bmu, wlv, blv,
             ue1, ce1, un1, cn1, un2, cn2, nrn, ce2, ue2):
    full = lambda shape: pl.BlockSpec(shape, lambda t: tuple(0 for _ in shape))
    vmem_args = [c2, x, eps, w1, b1, g1, bt1, w2, b2, g2, bt2, wmu, bmu, wlv,
                 blv, ue1, ce1, un1, cn1, un2, cn2, nrn]
    in_specs = [full(a.shape) for a in vmem_args]
    in_specs += [pl.BlockSpec(memory_space=pl.ANY),
                 pl.BlockSpec(memory_space=pl.ANY)]
    return pl.pallas_call(
        _tc_body,
        grid=(NSTEPS,),
        in_specs=in_specs,
        out_specs=pl.BlockSpec((N, N), lambda t: (0, 0)),
        out_shape=jax.ShapeDtypeStruct((N, N), F32),
        scratch_shapes=[
            pltpu.VMEM((1, HID), F32),
            pltpu.VMEM((1, N), F32),
            pltpu.VMEM((N, N), F32),
            pltpu.VMEM((2, SPAN, HID), F32),
            pltpu.VMEM((2, SPAN), F32),
            pltpu.VMEM((1, SPAN), F32),
            pltpu.SemaphoreType.DMA((2,)),
            pltpu.SemaphoreType.DMA((2,)),
        ],
        compiler_params=pltpu.CompilerParams(
            dimension_semantics=("arbitrary",)),
    )(c2, x, eps, w1, b1, g1, bt1, w2, b2, g2, bt2, wmu, bmu, wlv, blv,
      ue1, ce1, un1, cn1, un2, cn2, nrn, ce2, ue2)


def kernel(x, edge_index, num_real_nodes, eps, W1, b1, g1, bt1, W2, b2, g2,
           bt2, Wmu, bmu, Wlv, blv, Ue1, ce1, Ue2, ce2, Un1, cn1, Un2, cn2):
    c2 = _build_counts(edge_index).reshape(2, N, N)
    nrn = jnp.asarray(num_real_nodes, F32).reshape(1, 1)
    return _tc_call(c2, x, eps, W1, b1, g1, bt1, W2, b2, g2, bt2, Wmu, bmu,
                    Wlv, blv, Ue1, ce1, Un1, cn1, Un2, cn2, nrn, ce2, Ue2)
